# baseline (speedup 1.0000x reference)
import jax
import jax.numpy as jnp
from jax import lax
from jax.experimental import pallas as pl
from jax.experimental.pallas import tpu as pltpu
from jax.experimental.pallas import tpu_sc as plsc

NC = 2
NS = 16
L = 16
NW = NC * NS

B, SEQ, D = 4, 8192, 1024
N = B * SEQ
ROWS_PER_W = N // NW
K = 16
NCHUNK = ROWS_PER_W // K
NE = 4
NX = 3
PF = 2
U = 12
NFULL = (NCHUNK // U) * U


def _body(x_hbm, ids_hbm, emb_hbm, out_hbm, idx_v, xb, eb, gs, xs, osem):
  wid = lax.axis_index("s") * NC + lax.axis_index("c")
  base = wid * ROWS_PER_W
  pltpu.sync_copy(ids_hbm.at[pl.ds(base, ROWS_PER_W)], idx_v)

  GS = 8

  def start_in(g, je, jx):
    for h in range(K // GS):
      pltpu.async_copy(
          emb_hbm.at[idx_v.at[pl.ds(g * K + h * GS, GS)]],
          eb[je].at[pl.ds(h * GS, GS)], gs[je])
    pltpu.async_copy(x_hbm.at[pl.ds(base + g * K, K)], xb[jx], xs[jx])

  def wait_in(je, jx):
    for h in range(K // GS):
      pltpu.make_async_copy(
          x_hbm.at[pl.ds(0, GS)], eb[je].at[pl.ds(0, GS)], gs[je]).wait()
    pltpu.make_async_copy(x_hbm.at[pl.ds(0, K)], xb[jx], xs[jx]).wait()

  def wait_out(je):
    pltpu.make_async_copy(x_hbm.at[pl.ds(0, K)], eb[je], osem[je]).wait()

  def chunk_body(g, jmod):
    je, jx = jmod % NE, jmod % NX
    gp = g + PF
    jep, jxp = (jmod + PF) % NE, (jmod + PF) % NX

    @pl.when(gp < NCHUNK)
    def _prefetch():
      @pl.when(gp >= NE)
      def _drain():
        wait_out(jep)
      start_in(gp, jep, jxp)

    wait_in(je, jx)

    CU = 16

    def row(r, c2):
      def col(ci, c3):
        for u in range(CU):
          sl = pl.ds(ci * (CU * L) + u * L, L)
          plsc.addupdate(eb[je].at[r, sl], xb[jx][r, sl])
        return c3

      lax.fori_loop(0, D // L // CU, col, 0)
      return c2

    lax.fori_loop(0, K, row, 0)
    pltpu.async_copy(eb[je], out_hbm.at[pl.ds(base + g * K, K)], osem[je])

  for p in range(PF):
    start_in(p, p % NE, p % NX)

  def outer(go, carry):
    for j in range(U):
      chunk_body(go * U + j, j)
    return carry

  lax.fori_loop(0, NCHUNK // U, outer, 0)
  for g_tail in range(NFULL, NCHUNK):
    chunk_body(g_tail, g_tail % U)
  for j in range(NE):
    wait_out(j)


@jax.jit
def _run(x2d, ids, emb):
  mesh = plsc.VectorSubcoreMesh(
      core_axis_name="c", subcore_axis_name="s", num_cores=NC,
      num_subcores=NS)
  f = pl.kernel(
      _body,
      out_type=jax.ShapeDtypeStruct((N, D), jnp.float32),
      mesh=mesh,
      scratch_types=[
          pltpu.VMEM((ROWS_PER_W,), jnp.int32),
          [pltpu.VMEM((K, D), jnp.float32) for _ in range(NX)],
          [pltpu.VMEM((K, D), jnp.float32) for _ in range(NE)],
          [pltpu.SemaphoreType.DMA for _ in range(NE)],
          [pltpu.SemaphoreType.DMA for _ in range(NX)],
          [pltpu.SemaphoreType.DMA for _ in range(NE)],
      ],
  )
  return f(x2d, ids, emb)


def kernel(x, position_ids, pos_emb):
  x2d = x.reshape(N, D)
  ids = position_ids.astype(jnp.int32).reshape(N)
  out = _run(x2d, ids, pos_emb)
  return out.reshape(x.shape)


# speedup vs baseline: 1.8535x; 1.8535x over previous
import jax
import jax.numpy as jnp
from jax import lax
from jax.experimental import pallas as pl
from jax.experimental.pallas import tpu as pltpu
from jax.experimental.pallas import tpu_sc as plsc

NC = 2
NS = 16
L = 16
NW = NC * NS

B, SEQ, D = 4, 8192, 1024
N = B * SEQ
ROWS_PER_W = N // NW
K = 8
NCHUNK = ROWS_PER_W // K
NE = 4
NX = 4
PF = 2
U = 4
NFULL = (NCHUNK // U) * U


def _body(x_hbm, ids_hbm, emb_hbm, out_hbm, idx_v, xb, eb, gs, xs, osem):
  wid = lax.axis_index("s") * NC + lax.axis_index("c")
  base = wid * ROWS_PER_W
  pltpu.sync_copy(ids_hbm.at[pl.ds(base, ROWS_PER_W)], idx_v)

  GS = 8

  def start_in(g, je, jx):
    for h in range(K // GS):
      pltpu.async_copy(
          emb_hbm.at[idx_v.at[pl.ds(g * K + h * GS, GS)]],
          eb[je].at[pl.ds(h * GS, GS)], gs[je])
    pltpu.async_copy(x_hbm.at[pl.ds(base + g * K, K)], xb[jx], xs[jx])

  def wait_in(je, jx):
    for h in range(K // GS):
      pltpu.make_async_copy(
          x_hbm.at[pl.ds(0, GS)], eb[je].at[pl.ds(0, GS)], gs[je]).wait()
    pltpu.make_async_copy(x_hbm.at[pl.ds(0, K)], xb[jx], xs[jx]).wait()

  def wait_out(je):
    pltpu.make_async_copy(x_hbm.at[pl.ds(0, K)], eb[je], osem[je]).wait()

  def chunk_body(g, jmod):
    je, jx = jmod % NE, jmod % NX
    gp = g + PF
    jep, jxp = (jmod + PF) % NE, (jmod + PF) % NX

    @pl.when(gp < NCHUNK)
    def _prefetch():
      @pl.when(gp >= NE)
      def _drain():
        wait_out(jep)
      start_in(gp, jep, jxp)

    wait_in(je, jx)

    CU = 64

    def row(r, c2):
      def col(ci, c3):
        for u in range(CU):
          sl = pl.ds(ci * (CU * L) + u * L, L)
          plsc.addupdate(eb[je].at[r, sl], xb[jx][r, sl])
        return c3

      lax.fori_loop(0, D // L // CU, col, 0)
      return c2

    lax.fori_loop(0, K, row, 0)
    pltpu.async_copy(eb[je], out_hbm.at[pl.ds(base + g * K, K)], osem[je])

  for p in range(PF):
    start_in(p, p % NE, p % NX)

  def outer(go, carry):
    for j in range(U):
      chunk_body(go * U + j, j)
    return carry

  lax.fori_loop(0, NCHUNK // U, outer, 0)
  for g_tail in range(NFULL, NCHUNK):
    chunk_body(g_tail, g_tail % U)
  for j in range(NE):
    wait_out(j)


@jax.jit
def _run(x2d, ids, emb):
  mesh = plsc.VectorSubcoreMesh(
      core_axis_name="c", subcore_axis_name="s", num_cores=NC,
      num_subcores=NS)
  f = pl.kernel(
      _body,
      out_type=jax.ShapeDtypeStruct((N, D), jnp.float32),
      mesh=mesh,
      scratch_types=[
          pltpu.VMEM((ROWS_PER_W,), jnp.int32),
          [pltpu.VMEM((K, D), jnp.float32) for _ in range(NX)],
          [pltpu.VMEM((K, D), jnp.float32) for _ in range(NE)],
          [pltpu.SemaphoreType.DMA for _ in range(NE)],
          [pltpu.SemaphoreType.DMA for _ in range(NX)],
          [pltpu.SemaphoreType.DMA for _ in range(NE)],
      ],
  )
  return f(x2d, ids, emb)


def kernel(x, position_ids, pos_emb):
  x2d = x.reshape(N, D)
  ids = position_ids.astype(jnp.int32).reshape(N)
  out = _run(x2d, ids, pos_emb)
  return out.reshape(x.shape)
